# Initial kernel scaffold; baseline (speedup 1.0000x reference)
import jax
import jax.numpy as jnp
from jax.experimental import pallas as pl


def kernel(indices, embedding_matrix):
    raise NotImplementedError("write your pallas kernel here")


# speedup vs baseline: 1.7544x; 1.7544x over previous
import functools

import jax
import jax.numpy as jnp
from jax import lax
from jax.experimental import pallas as pl
from jax.experimental.pallas import tpu as pltpu
from jax.experimental.pallas import tpu_sc as plsc

BATCH = 16384
SEQ_LEN = 200
ENC_DIM = 32
TOTAL = BATCH * SEQ_LEN

_MESH = plsc.VectorSubcoreMesh(core_axis_name="c", subcore_axis_name="s")
NW = _MESH.num_cores * _MESH.num_subcores
PER_W = TOTAL // NW
IDX_MINOR = 128
K = 16
CHUNK = K * IDX_MINOR
NCH = PER_W // CHUNK


@functools.partial(
    pl.kernel,
    out_type=jax.ShapeDtypeStruct((TOTAL, ENC_DIM), jnp.float32),
    mesh=_MESH,
    scratch_types=[
        pltpu.VMEM((K, IDX_MINOR), jnp.int32),
        pltpu.VMEM((CHUNK, ENC_DIM), jnp.float32),
        pltpu.SemaphoreType.DMA,
    ],
    compiler_params=pltpu.CompilerParams(use_tc_tiling_on_sc=False),
)
def _sc_gather(idx_hbm, table_hbm, out_hbm, idx_v, rows_v, sem):
    wid = lax.axis_index("s") * _MESH.num_cores + lax.axis_index("c")
    base = wid * PER_W

    def chunk_body(g, _):
        row0 = base + g * CHUNK
        pltpu.sync_copy(idx_hbm.at[wid, g], idx_v)
        cps = [
            pltpu.async_copy(
                table_hbm.at[idx_v.at[j]],
                rows_v.at[pl.ds(j * IDX_MINOR, IDX_MINOR)],
                sem,
            )
            for j in range(K)
        ]
        for cp in cps:
            cp.wait()
        pltpu.sync_copy(rows_v, out_hbm.at[pl.ds(row0, CHUNK)])
        return ()

    lax.fori_loop(0, NCH, chunk_body, (), unroll=False)


def kernel(indices, embedding_matrix):
    idx = indices.astype(jnp.int32).reshape(NW, NCH, K, IDX_MINOR)
    out = _sc_gather(idx, embedding_matrix)
    return out.reshape(BATCH, SEQ_LEN, ENC_DIM)


# speedup vs baseline: 6.3925x; 3.6436x over previous
import functools

import jax
import jax.numpy as jnp
from jax import lax
from jax.experimental import pallas as pl
from jax.experimental.pallas import tpu as pltpu
from jax.experimental.pallas import tpu_sc as plsc

BATCH = 16384
SEQ_LEN = 200
ENC_DIM = 32
VOCAB = 25
HALF = SEQ_LEN // 2

_MESH = plsc.VectorSubcoreMesh(core_axis_name="c", subcore_axis_name="s")
NW = _MESH.num_cores * _MESH.num_subcores
BATCH_PER_W = BATCH // NW
RCHUNK = 8
K = RCHUNK * 2
NCH = BATCH_PER_W // RCHUNK


@functools.partial(
    pl.kernel,
    out_type=jax.ShapeDtypeStruct((BATCH, SEQ_LEN, ENC_DIM), jnp.float32),
    mesh=_MESH,
    scratch_types=[
        pltpu.VMEM((K, HALF), jnp.int32),
        pltpu.VMEM((RCHUNK, SEQ_LEN, ENC_DIM), jnp.float32),
        pltpu.VMEM_SHARED((VOCAB, ENC_DIM), jnp.float32),
        pltpu.SemaphoreType.DMA,
    ],
    compiler_params=pltpu.CompilerParams(use_tc_tiling_on_sc=False),
)
def _sc_gather(idx_hbm, table_hbm, out_hbm, idx_v, rows_v, table_s, sem):
    wid = lax.axis_index("s") * _MESH.num_cores + lax.axis_index("c")
    base = wid * BATCH_PER_W
    pltpu.sync_copy(table_hbm, table_s)

    def chunk_body(g, _):
        b0 = base + g * RCHUNK
        pltpu.sync_copy(idx_hbm.at[wid, g], idx_v)
        cps = [
            pltpu.async_copy(
                table_s.at[idx_v.at[j]],
                rows_v.at[j // 2, pl.ds((j % 2) * HALF, HALF)],
                sem,
            )
            for j in range(K)
        ]
        for cp in cps:
            cp.wait()
        pltpu.sync_copy(rows_v, out_hbm.at[pl.ds(b0, RCHUNK)])
        return ()

    lax.fori_loop(0, NCH, chunk_body, (), unroll=False)


def kernel(indices, embedding_matrix):
    idx = indices.astype(jnp.int32).reshape(NW, NCH, K, HALF)
    return _sc_gather(idx, embedding_matrix)


# speedup vs baseline: 16.9120x; 2.6456x over previous
import functools

import jax
import jax.numpy as jnp
from jax import lax
from jax.experimental import pallas as pl
from jax.experimental.pallas import tpu as pltpu
from jax.experimental.pallas import tpu_sc as plsc

BATCH = 16384
SEQ_LEN = 200
ENC_DIM = 32
VOCAB = 25

_MESH = plsc.VectorSubcoreMesh(core_axis_name="c", subcore_axis_name="s")
NW = _MESH.num_cores * _MESH.num_subcores
UNITS = SEQ_LEN * 4
UPW = UNITS // NW
BT = BATCH // 128
BLK = BT // 4


@functools.partial(
    pl.kernel,
    out_type=jax.ShapeDtypeStruct((SEQ_LEN, 4, 128, 8, 128), jnp.float32),
    mesh=_MESH,
    scratch_types=[
        pltpu.VMEM((BATCH,), jnp.int32),
        pltpu.VMEM((BATCH,), jnp.int32),
        pltpu.VMEM((BLK, 8, 128), jnp.float32),
        pltpu.VMEM((BLK, 8, 128), jnp.float32),
        pltpu.VMEM((VOCAB * ENC_DIM,), jnp.float32),
        pltpu.SemaphoreType.DMA,
        pltpu.SemaphoreType.DMA,
        pltpu.SemaphoreType.DMA,
        pltpu.SemaphoreType.DMA,
    ],
    compiler_params=pltpu.CompilerParams(
        use_tc_tiling_on_sc=False, needs_layout_passes=False),
)
def _sc_enc(idxT_hbm, tbl_hbm, out_hbm, idx_a, idx_b, st0, st1, tbl_v,
            sem_ia, sem_ib, sem_o0, sem_o1):
    wid = lax.axis_index("s") * _MESH.num_cores + lax.axis_index("c")
    u0 = wid * UPW
    pltpu.sync_copy(tbl_hbm, tbl_v)
    pltpu.async_copy(idxT_hbm.at[u0 // 4], idx_a, sem_ia)

    def do_unit(u, ibuf, isem, nbuf, nsem, prefetch):
        s = u // 4
        et = u % 4
        pltpu.make_async_copy(idxT_hbm.at[s], ibuf, isem).wait()
        if prefetch:
            pltpu.async_copy(idxT_hbm.at[(u + 1) // 4], nbuf, nsem)
        for blk in range(4):
            stage = st0 if blk % 2 == 0 else st1
            osem = sem_o0 if blk % 2 == 0 else sem_o1
            if blk < 2:
                @pl.when(u != u0)
                def _():
                    pltpu.make_async_copy(
                        stage, out_hbm.at[0, 0, pl.ds(0, BLK)], osem).wait()
            else:
                pltpu.make_async_copy(
                    stage, out_hbm.at[0, 0, pl.ds(0, BLK)], osem).wait()

            def bt_body(t, _):
                b0 = blk * (BLK * 128) + t * 128
                idxvs = [ibuf[pl.ds(b0 + l * 16, 16)] for l in range(8)]
                for ei in range(8):
                    ec = (et * 8 + ei) * VOCAB
                    for l in range(8):
                        val = plsc.load_gather(tbl_v, [idxvs[l] + ec])
                        stage[t, ei, pl.ds(l * 16, 16)] = val
                return ()

            lax.fori_loop(0, BLK, bt_body, (), unroll=False)
            pltpu.async_copy(
                stage, out_hbm.at[s, et, pl.ds(blk * BLK, BLK)], osem)

    def pair_body(p, _):
        ua = u0 + 2 * p
        do_unit(ua, idx_a, sem_ia, idx_b, sem_ib, True)
        do_unit(ua + 1, idx_b, sem_ib, idx_a, sem_ia, True)
        return ()

    lax.fori_loop(0, (UPW - 1) // 2, pair_body, (), unroll=False)
    do_unit(u0 + UPW - 1, idx_a, sem_ia, idx_b, sem_ib, False)
    pltpu.make_async_copy(st0, out_hbm.at[0, 0, pl.ds(0, BLK)], sem_o0).wait()
    pltpu.make_async_copy(st1, out_hbm.at[0, 0, pl.ds(0, BLK)], sem_o1).wait()


def kernel(indices, embedding_matrix):
    idxT = jnp.swapaxes(indices.astype(jnp.int32), 0, 1)
    tbl = jnp.swapaxes(embedding_matrix, 0, 1).reshape(-1)
    out5 = _sc_enc(idxT, tbl)
    return out5.transpose(2, 4, 0, 1, 3).reshape(BATCH, SEQ_LEN, ENC_DIM)


# speedup vs baseline: 59.9129x; 3.5426x over previous
import functools

import jax
import jax.numpy as jnp
from jax import lax
from jax.experimental import pallas as pl
from jax.experimental.pallas import tpu as pltpu
from jax.experimental.pallas import tpu_sc as plsc

BATCH = 16384
SEQ_LEN = 200
ENC_DIM = 32
VOCAB = 25
TBL_PAD = 832
_PIB = jax.lax.GatherScatterMode.PROMISE_IN_BOUNDS

_MESH = plsc.VectorSubcoreMesh(core_axis_name="c", subcore_axis_name="s")
NW = _MESH.num_cores * _MESH.num_subcores
UNITS = SEQ_LEN * 4
UPW = UNITS // NW
BT = BATCH // 128
BLK = BT // 4


@functools.partial(
    pl.kernel,
    out_type=jax.ShapeDtypeStruct((SEQ_LEN, 4, 128, 8, 128), jnp.float32),
    mesh=_MESH,
    scratch_types=[
        pltpu.VMEM((BATCH,), jnp.int32),
        pltpu.VMEM((BATCH,), jnp.int32),
        pltpu.VMEM((BLK, 8, 128), jnp.float32),
        pltpu.VMEM((BLK, 8, 128), jnp.float32),
        pltpu.VMEM((TBL_PAD,), jnp.float32),
        pltpu.SemaphoreType.DMA,
        pltpu.SemaphoreType.DMA,
        pltpu.SemaphoreType.DMA,
        pltpu.SemaphoreType.DMA,
    ],
    compiler_params=pltpu.CompilerParams(
        use_tc_tiling_on_sc=False, needs_layout_passes=False),
)
def _sc_enc(idxT_hbm, tbl_hbm, out_hbm, idx_a, idx_b, st0, st1, tbl_v,
            sem_ia, sem_ib, sem_o0, sem_o1):
    wid = lax.axis_index("s") * _MESH.num_cores + lax.axis_index("c")
    u0 = wid * UPW
    pltpu.sync_copy(tbl_hbm, tbl_v)
    pltpu.async_copy(idxT_hbm.at[u0 // 4], idx_a, sem_ia)

    def do_unit(u, ibuf, isem, nbuf, nsem, prefetch):
        s = u // 4
        et = u % 4
        pltpu.make_async_copy(idxT_hbm.at[s], ibuf, isem).wait()
        if prefetch:
            pltpu.async_copy(idxT_hbm.at[(u + 1) // 4], nbuf, nsem)
        rows = []
        for ei in range(8):
            e = et * 8 + ei
            rows.append((tbl_v[pl.ds(e * VOCAB, 16)],
                         tbl_v[pl.ds(e * VOCAB + 16, 16)]))
        for blk in range(4):
            stage = st0 if blk % 2 == 0 else st1
            osem = sem_o0 if blk % 2 == 0 else sem_o1
            if blk < 2:
                @pl.when(u != u0)
                def _():
                    pltpu.make_async_copy(
                        stage, out_hbm.at[0, 0, pl.ds(0, BLK)], osem).wait()
            else:
                pltpu.make_async_copy(
                    stage, out_hbm.at[0, 0, pl.ds(0, BLK)], osem).wait()

            def bt_body(t, _):
                b0 = blk * (BLK * 128) + t * 128
                idxvs = [ibuf[pl.ds(b0 + l * 16, 16)] for l in range(8)]
                lows = [iv < 16 for iv in idxvs]
                hids = [iv - 16 for iv in idxvs]
                for ei in range(8):
                    lo, hi = rows[ei]
                    for l in range(8):
                        val = jnp.where(
                            lows[l],
                            lo.at[idxvs[l]].get(mode=_PIB),
                            hi.at[hids[l]].get(mode=_PIB))
                        stage[t, ei, pl.ds(l * 16, 16)] = val
                return ()

            lax.fori_loop(0, BLK, bt_body, (), unroll=False)
            pltpu.async_copy(
                stage, out_hbm.at[s, et, pl.ds(blk * BLK, BLK)], osem)

    def pair_body(p, _):
        ua = u0 + 2 * p
        do_unit(ua, idx_a, sem_ia, idx_b, sem_ib, True)
        do_unit(ua + 1, idx_b, sem_ib, idx_a, sem_ia, True)
        return ()

    lax.fori_loop(0, (UPW - 1) // 2, pair_body, (), unroll=False)
    do_unit(u0 + UPW - 1, idx_a, sem_ia, idx_b, sem_ib, False)
    pltpu.make_async_copy(st0, out_hbm.at[0, 0, pl.ds(0, BLK)], sem_o0).wait()
    pltpu.make_async_copy(st1, out_hbm.at[0, 0, pl.ds(0, BLK)], sem_o1).wait()


def kernel(indices, embedding_matrix):
    idxT = jnp.swapaxes(indices.astype(jnp.int32), 0, 1)
    tbl = jnp.swapaxes(embedding_matrix, 0, 1).reshape(-1)
    tbl = jnp.pad(tbl, (0, TBL_PAD - VOCAB * ENC_DIM))
    out5 = _sc_enc(idxT, tbl)
    return out5.transpose(2, 4, 0, 1, 3).reshape(BATCH, SEQ_LEN, ENC_DIM)


# speedup vs baseline: 64.4793x; 1.0762x over previous
import functools

import jax
import jax.numpy as jnp
from jax import lax
from jax.experimental import pallas as pl
from jax.experimental.pallas import tpu as pltpu
from jax.experimental.pallas import tpu_sc as plsc

BATCH = 16384
SEQ_LEN = 200
ENC_DIM = 32
VOCAB = 25
TBL_PAD = 832
_PIB = jax.lax.GatherScatterMode.PROMISE_IN_BOUNDS

_MESH = plsc.VectorSubcoreMesh(core_axis_name="c", subcore_axis_name="s")
NW = _MESH.num_cores * _MESH.num_subcores
UNITS = SEQ_LEN * 4
UPW = UNITS // NW
BT = BATCH // 128
BLK = BT // 4


@functools.partial(
    pl.kernel,
    out_type=jax.ShapeDtypeStruct((SEQ_LEN, 4, 128, 8, 128), jnp.float32),
    mesh=_MESH,
    scratch_types=[
        pltpu.VMEM((128, 128), jnp.int32),
        pltpu.VMEM((128, 128), jnp.int32),
        pltpu.VMEM((BLK, 8, 128), jnp.float32),
        pltpu.VMEM((BLK, 8, 128), jnp.float32),
        pltpu.VMEM((TBL_PAD,), jnp.float32),
        pltpu.SemaphoreType.DMA,
        pltpu.SemaphoreType.DMA,
        pltpu.SemaphoreType.DMA,
        pltpu.SemaphoreType.DMA,
    ],
    compiler_params=pltpu.CompilerParams(
        use_tc_tiling_on_sc=False, needs_layout_passes=False),
)
def _sc_enc(idx4_hbm, tbl_hbm, out_hbm, idx_a, idx_b, st0, st1, tbl_v,
            sem_ia, sem_ib, sem_o0, sem_o1):
    wid = lax.axis_index("s") * _MESH.num_cores + lax.axis_index("c")
    u0 = wid * UPW

    def idx_copy(s, buf, sem):
        return pltpu.make_async_copy(
            idx4_hbm.at[s // 8, :, s % 8, :], buf, sem)

    pltpu.sync_copy(tbl_hbm, tbl_v)
    idx_copy(u0 // 4, idx_a, sem_ia).start()

    def do_unit(u, ibuf, isem, nbuf, nsem, prefetch):
        s = u // 4
        et = u % 4
        idx_copy(s, ibuf, isem).wait()
        if prefetch:
            idx_copy((u + 1) // 4, nbuf, nsem).start()
        rows = []
        for ei in range(8):
            e = et * 8 + ei
            rows.append((tbl_v[pl.ds(e * VOCAB, 16)],
                         tbl_v[pl.ds(e * VOCAB + 16, 16)]))
        for blk in range(4):
            stage = st0 if blk % 2 == 0 else st1
            osem = sem_o0 if blk % 2 == 0 else sem_o1
            if blk < 2:
                @pl.when(u != u0)
                def _():
                    pltpu.make_async_copy(
                        stage, out_hbm.at[0, 0, pl.ds(0, BLK)], osem).wait()
            else:
                pltpu.make_async_copy(
                    stage, out_hbm.at[0, 0, pl.ds(0, BLK)], osem).wait()

            def bt_body(t, _):
                bt = blk * BLK + t
                idxvs = [ibuf[bt, pl.ds(l * 16, 16)] for l in range(8)]
                lows = [iv < 16 for iv in idxvs]
                hids = [iv - 16 for iv in idxvs]
                for ei in range(8):
                    lo, hi = rows[ei]
                    for l in range(8):
                        val = jnp.where(
                            lows[l],
                            lo.at[idxvs[l]].get(mode=_PIB),
                            hi.at[hids[l]].get(mode=_PIB))
                        stage[t, ei, pl.ds(l * 16, 16)] = val
                return ()

            lax.fori_loop(0, BLK, bt_body, (), unroll=False)
            pltpu.async_copy(
                stage, out_hbm.at[s, et, pl.ds(blk * BLK, BLK)], osem)

    def pair_body(p, _):
        ua = u0 + 2 * p
        do_unit(ua, idx_a, sem_ia, idx_b, sem_ib, True)
        do_unit(ua + 1, idx_b, sem_ib, idx_a, sem_ia, True)
        return ()

    lax.fori_loop(0, (UPW - 1) // 2, pair_body, (), unroll=False)
    do_unit(u0 + UPW - 1, idx_a, sem_ia, idx_b, sem_ib, False)
    pltpu.make_async_copy(st0, out_hbm.at[0, 0, pl.ds(0, BLK)], sem_o0).wait()
    pltpu.make_async_copy(st1, out_hbm.at[0, 0, pl.ds(0, BLK)], sem_o1).wait()


def kernel(indices, embedding_matrix):
    idx4 = indices.astype(jnp.int32).reshape(128, 128, 25, 8)
    idx4 = idx4.transpose(2, 0, 3, 1)
    tbl = jnp.swapaxes(embedding_matrix, 0, 1).reshape(-1)
    tbl = jnp.pad(tbl, (0, TBL_PAD - VOCAB * ENC_DIM))
    out5 = _sc_enc(idx4, tbl)
    return out5.transpose(2, 4, 0, 1, 3).reshape(BATCH, SEQ_LEN, ENC_DIM)
